# Initial kernel scaffold; baseline (speedup 1.0000x reference)
import jax
import jax.numpy as jnp
from jax.experimental import pallas as pl


def kernel(left_features, edge_indices, edge_features, right_features, Wq, Wk, Wv, Wke, Wve, Wout, bn_gamma, bn_beta, W1, b1, W2, b2):
    raise NotImplementedError("write your pallas kernel here")


# speedup vs baseline: 73.9925x; 73.9925x over previous
import functools
import numpy as np
import jax
import jax.numpy as jnp
from jax import lax
from jax.experimental import pallas as pl
from jax.experimental.pallas import tpu as pltpu
from jax.experimental.pallas import tpu_sc as plsc

N = 10000
E = 320000
D = 128
ED = 4
H = 8
DH = 16

NW = 32
C = 32
NCH = 313
PERW = C * NCH
EPAD = NW * PERW
NPAD = 10016
RPS = NPAD // 16


def _perm128() -> np.ndarray:
    p = np.zeros(128, dtype=np.int32)
    for j in range(8):
        for l in range(16):
            p[j * 16 + l] = (l * 16 + 2 * j) if l < 8 else ((15 - l) * 16 + 2 * j + 1)
    return p


_PERM = _perm128()


_RB = 2504
_Z = np.int32(0)


def _tables_body(right_ref, left_ref, wd_ref, ws_ref, dtab_ref, stab_ref):
    dtab_ref[...] = jnp.dot(right_ref[...], wd_ref[...],
                            preferred_element_type=jnp.float32,
                            precision=lax.Precision.HIGHEST)
    stab_ref[...] = jnp.dot(left_ref[...], ws_ref[...],
                            preferred_element_type=jnp.float32,
                            precision=lax.Precision.HIGHEST)


def _make_tables(right_p, left_p, Wd, Ws):
    return pl.pallas_call(
        _tables_body,
        grid=(NPAD // _RB,),
        in_specs=[pl.BlockSpec((_RB, D), lambda i: (i, _Z)),
                  pl.BlockSpec((_RB, D), lambda i: (i, _Z)),
                  pl.BlockSpec((D, 160), lambda i: (_Z, _Z)),
                  pl.BlockSpec((D, 256), lambda i: (_Z, _Z))],
        out_specs=(pl.BlockSpec((_RB, 160), lambda i: (i, _Z)),
                   pl.BlockSpec((_RB, 256), lambda i: (i, _Z))),
        out_shape=(jax.ShapeDtypeStruct((NPAD, 160), jnp.float32),
                   jax.ShapeDtypeStruct((NPAD, 256), jnp.float32)),
    )(right_p, left_p, Wd, Ws)


def _edge_kernel(dtab, stab, efx, srci, dsti, outm, outs,
                 accm, accs, sidx, didx, drows, srows, efb, msgm, msgs,
                 sem0, sem1):
    i32 = jnp.int32
    cid = lax.axis_index("c")
    sid = lax.axis_index("s")
    wid = sid * i32(2) + cid

    def _zrow(e, _):
        for j in range(8):
            msgm[e, pl.ds(16 * j, 16)] = jnp.zeros((16,), jnp.float32)
        for j in range(3):
            msgs[e, pl.ds(16 * j, 16)] = jnp.zeros((16,), jnp.float32)
        return _
    lax.fori_loop(jnp.int32(0), jnp.int32(C), _zrow, 0)
    row0 = sid * i32(RPS)
    nfull = RPS // C
    rem = RPS - nfull * C
    for i in range(nfull):
        pltpu.sync_copy(msgm, accm.at[pl.ds(row0 + i * C, C)])
        pltpu.sync_copy(msgs, accs.at[pl.ds(row0 + i * C, C)])
    if rem:
        pltpu.sync_copy(msgm.at[pl.ds(0, rem)],
                        accm.at[pl.ds(row0 + nfull * C, rem)])
        pltpu.sync_copy(msgs.at[pl.ds(0, rem)],
                        accs.at[pl.ds(row0 + nfull * C, rem)])
    plsc.subcore_barrier()

    base0 = wid * i32(PERW)

    def _chunk(g, _):
        base = base0 + g * i32(C)
        pltpu.sync_copy(srci.at[pl.ds(base, C)], sidx)
        pltpu.sync_copy(dsti.at[pl.ds(base, C)], didx)
        cps = pltpu.async_copy(stab.at[sidx], srows, sem0)
        cpd = pltpu.async_copy(dtab.at[didx], drows, sem1)
        pltpu.sync_copy(efx.at[pl.ds(base, C)], efb)
        cps.wait()
        cpd.wait()

        def _edge(e, _):
            s = drows[e, pl.ds(0, 16)] * srows[e, pl.ds(0, 16)]
            for j in range(1, 8):
                s = s + drows[e, pl.ds(16 * j, 16)] * srows[e, pl.ds(16 * j, 16)]
            qk = s + lax.rev(s, (0,))
            e0 = efb[e, pl.ds(0, 16)]
            e1 = efb[e, pl.ds(16, 16)]
            t = drows[e, pl.ds(128, 16)] * e0 + drows[e, pl.ds(144, 16)] * e1
            et = t + lax.rev(t, (0,))
            ex = jnp.exp((qk + et) * 0.25)
            for j in range(8):
                msgm[e, pl.ds(16 * j, 16)] = srows[e, pl.ds(128 + 16 * j, 16)] * ex
            msgs[e, pl.ds(0, 16)] = ex * e0
            msgs[e, pl.ds(16, 16)] = ex * e1
            msgs[e, pl.ds(32, 16)] = ex
            return _
        lax.fori_loop(jnp.int32(0), jnp.int32(C), _edge, 0)
        pltpu.sync_copy(msgm, accm.at[didx], add=True)
        pltpu.sync_copy(msgs, accs.at[didx], add=True)
        return _

    lax.fori_loop(jnp.int32(0), jnp.int32(NCH), _chunk, 0)
    plsc.subcore_barrier()

    for i in range(nfull):
        pltpu.sync_copy(accm.at[pl.ds(row0 + i * C, C)],
                        outm.at[cid, pl.ds(row0 + i * C, C)])
        pltpu.sync_copy(accs.at[pl.ds(row0 + i * C, C)],
                        outs.at[cid, pl.ds(row0 + i * C, C)])
    if rem:
        pltpu.sync_copy(accm.at[pl.ds(row0 + nfull * C, rem)],
                        outm.at[cid, pl.ds(row0 + nfull * C, rem)])
        pltpu.sync_copy(accs.at[pl.ds(row0 + nfull * C, rem)],
                        outs.at[cid, pl.ds(row0 + nfull * C, rem)])


_edge_pass = functools.partial(
    pl.kernel,
    out_type=(jax.ShapeDtypeStruct((2, NPAD, 128), jnp.float32),
              jax.ShapeDtypeStruct((2, NPAD, 48), jnp.float32)),
    mesh=plsc.VectorSubcoreMesh(core_axis_name="c", subcore_axis_name="s"),
    compiler_params=pltpu.CompilerParams(use_tc_tiling_on_sc=False),
    scratch_types=[
        pltpu.VMEM_SHARED((NPAD, 128), jnp.float32),
        pltpu.VMEM_SHARED((NPAD, 48), jnp.float32),
        pltpu.VMEM((C,), jnp.int32),
        pltpu.VMEM((C,), jnp.int32),
        pltpu.VMEM((C, 160), jnp.float32),
        pltpu.VMEM((C, 256), jnp.float32),
        pltpu.VMEM((C, 32), jnp.float32),
        pltpu.VMEM((C, 128), jnp.float32),
        pltpu.VMEM((C, 48), jnp.float32),
        pltpu.SemaphoreType.DMA,
        pltpu.SemaphoreType.DMA,
    ],
)(_edge_kernel)


def _proj_body(accm_ref, accs_ref, wbig_ref, out_ref):
    accm = accm_ref[0] + accm_ref[1]
    accs = accs_ref[0] + accs_ref[1]
    den = accs[:, 32:48]
    rec = jnp.where(den > 0.0, 1.0 / den, 0.0)
    scaled = jnp.concatenate(
        [accm * jnp.concatenate([rec] * 8, axis=1),
         accs[:, :32] * jnp.concatenate([rec] * 2, axis=1)], axis=1)
    out_ref[...] = jnp.dot(scaled, wbig_ref[...],
                           preferred_element_type=jnp.float32,
                           precision=lax.Precision.HIGHEST)


def _mlp_body(out_ref, right_ref, g_ref, b_ref,
              w1a_ref, w1b_ref, b1_ref, w2t_ref, b2_ref, y_ref):
    out = out_ref[:N, :]
    mean = jnp.mean(out, axis=0, keepdims=True)
    dcen = out - mean
    var = jnp.mean(dcen * dcen, axis=0, keepdims=True)
    outb = dcen * lax.rsqrt(var + 1e-5) * g_ref[...] + b_ref[...]
    h1 = jnp.dot(outb, w1a_ref[...], preferred_element_type=jnp.float32,
                 precision=lax.Precision.HIGHEST)
    h1 = h1 + jnp.dot(right_ref[...], w1b_ref[...],
                      preferred_element_type=jnp.float32,
                      precision=lax.Precision.HIGHEST)
    h1 = jnp.maximum(h1 + b1_ref[...], 0.0)
    y_ref[...] = jnp.dot(h1, w2t_ref[...],
                         preferred_element_type=jnp.float32,
                         precision=lax.Precision.HIGHEST) + b2_ref[...]


def _epilogue(accm, accs, right, Wbig, gamma, beta, W1a, W1b, b1, W2t, b2):
    out = pl.pallas_call(
        _proj_body,
        grid=(NPAD // _RB,),
        in_specs=[pl.BlockSpec((2, _RB, 128), lambda i: (_Z, i, _Z)),
                  pl.BlockSpec((2, _RB, 48), lambda i: (_Z, i, _Z)),
                  pl.BlockSpec((160, D), lambda i: (_Z, _Z))],
        out_specs=pl.BlockSpec((_RB, D), lambda i: (i, _Z)),
        out_shape=jax.ShapeDtypeStruct((NPAD, D), jnp.float32),
    )(accm, accs, Wbig)
    return pl.pallas_call(
        _mlp_body,
        out_shape=jax.ShapeDtypeStruct((N, D), jnp.float32),
    )(out, right, gamma.reshape(1, D), beta.reshape(1, D),
      W1a, W1b, b1.reshape(1, D), W2t, b2.reshape(1, D))


def kernel(left_features, edge_indices, edge_features, right_features,
           Wq, Wk, Wv, Wke, Wve, Wout, bn_gamma, bn_beta, W1, b1, W2, b2):
    f32 = jnp.float32
    left = left_features.astype(f32)
    right = right_features.astype(f32)

    perm = _PERM
    Mq = jnp.einsum("hkd,hkc->dhc", Wq.reshape(H, DH, D), Wke.reshape(H, DH, ED))
    cols = []
    for p in range(2):
        for l in range(16):
            h = l if l < 8 else 15 - l
            c = 2 * p + (0 if l < 8 else 1)
            cols.append(Mq[:, h, c])
    M_lay = jnp.stack(cols, axis=1)
    Wd = jnp.concatenate([Wq.T[:, perm], M_lay], axis=1)
    Ws = jnp.concatenate([Wk.T[:, perm], Wv.T[:, perm]], axis=1)

    Gq = jnp.einsum("hkc,ohk->hco", Wve.reshape(H, DH, ED), Wout.reshape(D, H, DH))
    rows = []
    for p in range(2):
        for l in range(16):
            h = l if l < 8 else 15 - l
            c = 2 * p + (0 if l < 8 else 1)
            rows.append(Gq[h, c])
    G_lay = jnp.stack(rows, axis=0)
    Wbig = jnp.concatenate([Wout.T[perm], G_lay], axis=0)
    W1t = W1.T
    W1a, W1b = W1t[:D], W1t[D:]
    W2t = W2.T

    right_p = jnp.pad(right, ((0, NPAD - N), (0, 0)))
    left_p = jnp.pad(left, ((0, NPAD - N), (0, 0)))
    srci = jnp.full((EPAD,), N, jnp.int32).at[:E].set(
        edge_indices[0].astype(jnp.int32))
    dsti = jnp.full((EPAD,), N, jnp.int32).at[:E].set(
        edge_indices[1].astype(jnp.int32))
    ef = edge_features.astype(f32)
    efx = jnp.pad(
        jnp.broadcast_to(ef[:, :, None], (E, ED, 8)).reshape(E, 32),
        ((0, EPAD - E), (0, 0)))

    dtab, stab = _make_tables(right_p, left_p, Wd, Ws)
    accm, accs = _edge_pass(dtab, stab, efx, srci, dsti)
    y = _epilogue(accm, accs, right, Wbig,
                  bn_gamma.astype(f32), bn_beta.astype(f32),
                  W1a, W1b, b1.astype(f32), W2t, b2.astype(f32))
    return y.astype(jnp.float64)


# speedup vs baseline: 93.6246x; 1.2653x over previous
import functools
import numpy as np
import jax
import jax.numpy as jnp
from jax import lax
from jax.experimental import pallas as pl
from jax.experimental.pallas import tpu as pltpu
from jax.experimental.pallas import tpu_sc as plsc

N = 10000
E = 320000
D = 128
ED = 4
H = 8
DH = 16

NW = 32
C = 16
S = 8
NCH = 640
PERW = C * NCH
EPAD = NW * PERW
EPAD2 = EPAD + 2 * S * C
NPAD = 10016
RPS = NPAD // 16


def _perm128() -> np.ndarray:
    p = np.zeros(128, dtype=np.int32)
    for j in range(8):
        for l in range(16):
            p[j * 16 + l] = (l * 16 + 2 * j) if l < 8 else ((15 - l) * 16 + 2 * j + 1)
    return p


_PERM = _perm128()


_RB = 2504
_Z = np.int32(0)


def _tables_body(right_ref, left_ref, wd_ref, ws_ref, dtab_ref, stab_ref):
    dtab_ref[...] = jnp.dot(right_ref[...], wd_ref[...],
                            preferred_element_type=jnp.float32,
                            precision=lax.Precision.HIGHEST)
    stab_ref[...] = jnp.dot(left_ref[...], ws_ref[...],
                            preferred_element_type=jnp.float32,
                            precision=lax.Precision.HIGHEST)


def _make_tables(right_p, left_p, Wd, Ws):
    return pl.pallas_call(
        _tables_body,
        grid=(NPAD // _RB,),
        in_specs=[pl.BlockSpec((_RB, D), lambda i: (i, _Z)),
                  pl.BlockSpec((_RB, D), lambda i: (i, _Z)),
                  pl.BlockSpec((D, 160), lambda i: (_Z, _Z)),
                  pl.BlockSpec((D, 256), lambda i: (_Z, _Z))],
        out_specs=(pl.BlockSpec((_RB, 160), lambda i: (i, _Z)),
                   pl.BlockSpec((_RB, 256), lambda i: (i, _Z))),
        out_shape=(jax.ShapeDtypeStruct((NPAD, 160), jnp.float32),
                   jax.ShapeDtypeStruct((NPAD, 256), jnp.float32)),
    )(right_p, left_p, Wd, Ws)


def _edge_kernel(dtab, stab, efx, srci, dsti, outm, outs,
                 accm, accs,
                 sidx0, didx0, dr0, sr0, ef0,
                 sidx1, didx1, dr1, sr1, ef1,
                 msgm, msgs,
                 semi0, semr0, semi1, semr1):
    i32 = jnp.int32
    cid = lax.axis_index("c")
    sid = lax.axis_index("s")
    wid = sid * i32(2) + cid

    def _zrow(e, _):
        for j in range(8):
            msgm[e, pl.ds(16 * j, 16)] = jnp.zeros((16,), jnp.float32)
        for j in range(3):
            msgs[e, pl.ds(16 * j, 16)] = jnp.zeros((16,), jnp.float32)
        return _
    lax.fori_loop(jnp.int32(0), jnp.int32(C), _zrow, 0)
    row0 = sid * i32(RPS)
    nfull = RPS // C
    rem = RPS - nfull * C
    for i in range(nfull):
        pltpu.sync_copy(msgm, accm.at[pl.ds(row0 + i * C, C)])
        pltpu.sync_copy(msgs, accs.at[pl.ds(row0 + i * C, C)])
    if rem:
        pltpu.sync_copy(msgm.at[pl.ds(jnp.int32(0), rem)],
                        accm.at[pl.ds(row0 + nfull * C, rem)])
        pltpu.sync_copy(msgs.at[pl.ds(jnp.int32(0), rem)],
                        accs.at[pl.ds(row0 + nfull * C, rem)])
    plsc.subcore_barrier()

    base0 = wid * i32(PERW)
    ibufs = ((sidx0, didx0, semi0), (sidx1, didx1, semi1))
    rbufs = ((dr0, sr0, ef0, semr0), (dr1, sr1, ef1, semr1))

    chbase = wid * i32(NCH)

    def _issue_idx(m, ib):
        base = chbase + m * i32(S)
        pltpu.async_copy(srci.at[pl.ds(base, S)], ib[0], ib[2])
        pltpu.async_copy(dsti.at[pl.ds(base, S)], ib[1], ib[2])

    def _wait_idx(ib):
        pltpu.make_async_copy(srci.at[pl.ds(jnp.int32(0), S)], ib[0], ib[2]).wait()
        pltpu.make_async_copy(dsti.at[pl.ds(jnp.int32(0), S)], ib[1], ib[2]).wait()

    def _issue_rows(g, k, ib, rb):
        base = base0 + g * i32(C)
        kk = jnp.int32(k)
        pltpu.async_copy(dtab.at[ib[1].at[kk]], rb[0], rb[3])
        pltpu.async_copy(stab.at[ib[0].at[kk]], rb[1], rb[3])
        pltpu.async_copy(efx.at[pl.ds(base, C)], rb[2], rb[3])

    def _wait_rows(ib, rb):
        pltpu.make_async_copy(dtab.at[ib[1].at[jnp.int32(0)]], rb[0], rb[3]).wait()
        pltpu.make_async_copy(stab.at[ib[0].at[jnp.int32(0)]], rb[1], rb[3]).wait()
        pltpu.make_async_copy(efx.at[pl.ds(jnp.int32(0), C)], rb[2], rb[3]).wait()

    _issue_idx(i32(0), ibufs[0])
    _issue_idx(i32(1), ibufs[1])
    _wait_idx(ibufs[0])
    _issue_rows(i32(0), jnp.int32(0), ibufs[0], rbufs[0])

    def _compute_chunk(rb, didx_row):
        drows, srows, efb = rb[0], rb[1], rb[2]

        def _edge(e, _):
            s = drows[e, pl.ds(0, 16)] * srows[e, pl.ds(0, 16)]
            for j in range(1, 8):
                s = s + drows[e, pl.ds(16 * j, 16)] * srows[e, pl.ds(16 * j, 16)]
            qk = s + lax.rev(s, (0,))
            e0 = efb[e, pl.ds(0, 16)]
            e1 = efb[e, pl.ds(16, 16)]
            t2 = drows[e, pl.ds(128, 16)] * e0 + drows[e, pl.ds(144, 16)] * e1
            et = t2 + lax.rev(t2, (0,))
            ex = jnp.exp((qk + et) * 0.25)
            for j in range(8):
                msgm[e, pl.ds(16 * j, 16)] = srows[e, pl.ds(128 + 16 * j, 16)] * ex
            msgs[e, pl.ds(0, 16)] = ex * e0
            msgs[e, pl.ds(16, 16)] = ex * e1
            msgs[e, pl.ds(32, 16)] = ex
            return _
        lax.fori_loop(jnp.int32(0), jnp.int32(C), _edge, 0)
        pltpu.sync_copy(msgm, accm.at[didx_row], add=True)
        pltpu.sync_copy(msgs, accs.at[didx_row], add=True)

    def _superpair(t, _):
        for u in range(2):
            m = t * i32(2) + i32(u)
            ib = ibufs[u]
            ibn = ibufs[1 - u]
            g0 = m * i32(S)
            for k in range(S):
                rb = rbufs[k & 1]
                rbn = rbufs[1 - (k & 1)]
                _wait_rows(ib, rb)
                if k < S - 1:
                    _issue_rows(g0 + i32(k + 1), jnp.int32(k + 1), ib, rbn)
                    _compute_chunk(rb, ib[1].at[jnp.int32(k)])
                else:
                    _wait_idx(ibn)
                    _issue_rows(g0 + i32(S), jnp.int32(0), ibn, rbn)
                    _compute_chunk(rb, ib[1].at[jnp.int32(k)])
                    _issue_idx(m + i32(2), ib)
        return _

    lax.fori_loop(jnp.int32(0), jnp.int32(NCH // (2 * S)), _superpair, 0)
    _wait_rows(ibufs[0], rbufs[0])
    _wait_idx(ibufs[1])
    plsc.subcore_barrier()

    for i in range(nfull):
        pltpu.sync_copy(accm.at[pl.ds(row0 + i * C, C)],
                        outm.at[cid, pl.ds(row0 + i * C, C)])
        pltpu.sync_copy(accs.at[pl.ds(row0 + i * C, C)],
                        outs.at[cid, pl.ds(row0 + i * C, C)])
    if rem:
        pltpu.sync_copy(accm.at[pl.ds(row0 + nfull * C, rem)],
                        outm.at[cid, pl.ds(row0 + nfull * C, rem)])
        pltpu.sync_copy(accs.at[pl.ds(row0 + nfull * C, rem)],
                        outs.at[cid, pl.ds(row0 + nfull * C, rem)])


_edge_pass = functools.partial(
    pl.kernel,
    out_type=(jax.ShapeDtypeStruct((2, NPAD, 128), jnp.float32),
              jax.ShapeDtypeStruct((2, NPAD, 48), jnp.float32)),
    mesh=plsc.VectorSubcoreMesh(core_axis_name="c", subcore_axis_name="s"),
    compiler_params=pltpu.CompilerParams(use_tc_tiling_on_sc=False),
    scratch_types=[
        pltpu.VMEM_SHARED((NPAD, 128), jnp.float32),
        pltpu.VMEM_SHARED((NPAD, 48), jnp.float32),
        pltpu.VMEM((S, C), jnp.int32),
        pltpu.VMEM((S, C), jnp.int32),
        pltpu.VMEM((C, 160), jnp.float32),
        pltpu.VMEM((C, 256), jnp.float32),
        pltpu.VMEM((C, 32), jnp.float32),
        pltpu.VMEM((S, C), jnp.int32),
        pltpu.VMEM((S, C), jnp.int32),
        pltpu.VMEM((C, 160), jnp.float32),
        pltpu.VMEM((C, 256), jnp.float32),
        pltpu.VMEM((C, 32), jnp.float32),
        pltpu.VMEM((C, 128), jnp.float32),
        pltpu.VMEM((C, 48), jnp.float32),
        pltpu.SemaphoreType.DMA,
        pltpu.SemaphoreType.DMA,
        pltpu.SemaphoreType.DMA,
        pltpu.SemaphoreType.DMA,
    ],
)(_edge_kernel)


def _proj_body(accm_ref, accs_ref, wbig_ref, out_ref):
    accm = accm_ref[0] + accm_ref[1]
    accs = accs_ref[0] + accs_ref[1]
    den = accs[:, 32:48]
    rec = jnp.where(den > 0.0, 1.0 / den, 0.0)
    scaled = jnp.concatenate(
        [accm * jnp.concatenate([rec] * 8, axis=1),
         accs[:, :32] * jnp.concatenate([rec] * 2, axis=1)], axis=1)
    out_ref[...] = jnp.dot(scaled, wbig_ref[...],
                           preferred_element_type=jnp.float32,
                           precision=lax.Precision.HIGHEST)


def _mlp_body(out_ref, right_ref, g_ref, b_ref,
              w1a_ref, w1b_ref, b1_ref, w2t_ref, b2_ref, y_ref):
    out = out_ref[:N, :]
    mean = jnp.mean(out, axis=0, keepdims=True)
    dcen = out - mean
    var = jnp.mean(dcen * dcen, axis=0, keepdims=True)
    outb = dcen * lax.rsqrt(var + 1e-5) * g_ref[...] + b_ref[...]
    h1 = jnp.dot(outb, w1a_ref[...], preferred_element_type=jnp.float32,
                 precision=lax.Precision.HIGHEST)
    h1 = h1 + jnp.dot(right_ref[...], w1b_ref[...],
                      preferred_element_type=jnp.float32,
                      precision=lax.Precision.HIGHEST)
    h1 = jnp.maximum(h1 + b1_ref[...], 0.0)
    y_ref[...] = jnp.dot(h1, w2t_ref[...],
                         preferred_element_type=jnp.float32,
                         precision=lax.Precision.HIGHEST) + b2_ref[...]


def _epilogue(accm, accs, right, Wbig, gamma, beta, W1a, W1b, b1, W2t, b2):
    out = pl.pallas_call(
        _proj_body,
        grid=(NPAD // _RB,),
        in_specs=[pl.BlockSpec((2, _RB, 128), lambda i: (_Z, i, _Z)),
                  pl.BlockSpec((2, _RB, 48), lambda i: (_Z, i, _Z)),
                  pl.BlockSpec((160, D), lambda i: (_Z, _Z))],
        out_specs=pl.BlockSpec((_RB, D), lambda i: (i, _Z)),
        out_shape=jax.ShapeDtypeStruct((NPAD, D), jnp.float32),
    )(accm, accs, Wbig)
    return pl.pallas_call(
        _mlp_body,
        out_shape=jax.ShapeDtypeStruct((N, D), jnp.float32),
    )(out, right, gamma.reshape(1, D), beta.reshape(1, D),
      W1a, W1b, b1.reshape(1, D), W2t, b2.reshape(1, D))


def kernel(left_features, edge_indices, edge_features, right_features,
           Wq, Wk, Wv, Wke, Wve, Wout, bn_gamma, bn_beta, W1, b1, W2, b2):
    f32 = jnp.float32
    left = left_features.astype(f32)
    right = right_features.astype(f32)

    perm = _PERM
    Mq = jnp.einsum("hkd,hkc->dhc", Wq.reshape(H, DH, D), Wke.reshape(H, DH, ED))
    cols = []
    for p in range(2):
        for l in range(16):
            h = l if l < 8 else 15 - l
            c = 2 * p + (0 if l < 8 else 1)
            cols.append(Mq[:, h, c])
    M_lay = jnp.stack(cols, axis=1)
    Wd = jnp.concatenate([Wq.T[:, perm], M_lay], axis=1)
    Ws = jnp.concatenate([Wk.T[:, perm], Wv.T[:, perm]], axis=1)

    Gq = jnp.einsum("hkc,ohk->hco", Wve.reshape(H, DH, ED), Wout.reshape(D, H, DH))
    rows = []
    for p in range(2):
        for l in range(16):
            h = l if l < 8 else 15 - l
            c = 2 * p + (0 if l < 8 else 1)
            rows.append(Gq[h, c])
    G_lay = jnp.stack(rows, axis=0)
    Wbig = jnp.concatenate([Wout.T[perm], G_lay], axis=0)
    W1t = W1.T
    W1a, W1b = W1t[:D], W1t[D:]
    W2t = W2.T

    right_p = jnp.pad(right, ((0, NPAD - N), (0, 0)))
    left_p = jnp.pad(left, ((0, NPAD - N), (0, 0)))
    srci = jnp.full((EPAD2,), N, jnp.int32).at[:E].set(
        edge_indices[0].astype(jnp.int32)).reshape(-1, C)
    dsti = jnp.full((EPAD2,), N, jnp.int32).at[:E].set(
        edge_indices[1].astype(jnp.int32)).reshape(-1, C)
    ef = edge_features.astype(f32)
    efx = jnp.pad(
        jnp.broadcast_to(ef[:, :, None], (E, ED, 8)).reshape(E, 32),
        ((0, EPAD2 - E), (0, 0)))

    dtab, stab = _make_tables(right_p, left_p, Wd, Ws)
    accm, accs = _edge_pass(dtab, stab, efx, srci, dsti)
    y = _epilogue(accm, accs, right, Wbig,
                  bn_gamma.astype(f32), bn_beta.astype(f32),
                  W1a, W1b, b1.astype(f32), W2t, b2.astype(f32))
    return y.astype(jnp.float64)


# speedup vs baseline: 118.0090x; 1.2604x over previous
import functools
import numpy as np
import jax
import jax.numpy as jnp
from jax import lax
from jax.experimental import pallas as pl
from jax.experimental.pallas import tpu as pltpu
from jax.experimental.pallas import tpu_sc as plsc

N = 10000
E = 320000
D = 128
ED = 4
H = 8
DH = 16

NW = 32
C = 32
S = 8
NCH = 320
PERW = C * NCH
EPAD = NW * PERW
EPAD2 = EPAD + 2 * S * C
NPAD = 10016
RPS = NPAD // 16


def _perm128() -> np.ndarray:
    p = np.zeros(128, dtype=np.int32)
    for j in range(8):
        for l in range(16):
            p[j * 16 + l] = (l * 16 + 2 * j) if l < 8 else ((15 - l) * 16 + 2 * j + 1)
    return p


_PERM = _perm128()


def _ilv(n: int) -> np.ndarray:
    p = np.zeros(n, dtype=np.int32)
    for b in range(n // 32):
        for i in range(16):
            p[32 * b + 2 * i] = 32 * b + i
            p[32 * b + 2 * i + 1] = 32 * b + 16 + i
    return p


_ILV160 = _ilv(160)
_ILV256 = _ilv(256)


_RB = 5008
_Z = np.int32(0)


def _tables_body(right_ref, left_ref, wd_ref, ws_ref, dtab_ref, stab_ref):
    dtab_ref[...] = jnp.dot(right_ref[...], wd_ref[...],
                            preferred_element_type=jnp.float32,
                            precision=lax.Precision.HIGHEST).astype(jnp.bfloat16)
    stab_ref[...] = jnp.dot(left_ref[...], ws_ref[...],
                            preferred_element_type=jnp.float32,
                            precision=lax.Precision.HIGHEST).astype(jnp.bfloat16)


def _make_tables(right_p, left_p, Wd, Ws):
    return pl.pallas_call(
        _tables_body,
        grid=(NPAD // _RB,),
        in_specs=[pl.BlockSpec((_RB, D), lambda i: (i, _Z)),
                  pl.BlockSpec((_RB, D), lambda i: (i, _Z)),
                  pl.BlockSpec((D, 160), lambda i: (_Z, _Z)),
                  pl.BlockSpec((D, 256), lambda i: (_Z, _Z))],
        out_specs=(pl.BlockSpec((_RB, 160), lambda i: (i, _Z)),
                   pl.BlockSpec((_RB, 256), lambda i: (i, _Z))),
        out_shape=(jax.ShapeDtypeStruct((NPAD, 160), jnp.bfloat16),
                   jax.ShapeDtypeStruct((NPAD, 256), jnp.bfloat16)),
    )(right_p, left_p, Wd, Ws)


def _edge_kernel(dtab, stab, efx, srci, dsti, outm, outs,
                 accm, accs,
                 sidx0, didx0, dr0, sr0, ef0,
                 sidx1, didx1, dr1, sr1, ef1,
                 msgm, msgs,
                 semi0, semr0, semi1, semr1):
    i32 = jnp.int32
    cid = lax.axis_index("c")
    sid = lax.axis_index("s")
    wid = sid * i32(2) + cid

    def _zrow(e, _):
        for j in range(8):
            msgm[e, pl.ds(16 * j, 16)] = jnp.zeros((16,), jnp.float32)
        for j in range(3):
            msgs[e, pl.ds(16 * j, 16)] = jnp.zeros((16,), jnp.float32)
        return _
    lax.fori_loop(jnp.int32(0), jnp.int32(C), _zrow, 0)
    row0 = sid * i32(RPS)
    nfull = RPS // C
    rem = RPS - nfull * C
    for i in range(nfull):
        pltpu.sync_copy(msgm, accm.at[pl.ds(row0 + i * C, C)])
        pltpu.sync_copy(msgs, accs.at[pl.ds(row0 + i * C, C)])
    if rem:
        pltpu.sync_copy(msgm.at[pl.ds(jnp.int32(0), rem)],
                        accm.at[pl.ds(row0 + nfull * C, rem)])
        pltpu.sync_copy(msgs.at[pl.ds(jnp.int32(0), rem)],
                        accs.at[pl.ds(row0 + nfull * C, rem)])
    plsc.subcore_barrier()

    base0 = wid * i32(PERW)
    ibufs = ((sidx0, didx0, semi0), (sidx1, didx1, semi1))
    rbufs = ((dr0, sr0, ef0, semr0), (dr1, sr1, ef1, semr1))

    chbase = wid * i32(NCH)

    def _issue_idx(m, ib):
        base = chbase + m * i32(S)
        pltpu.async_copy(srci.at[pl.ds(base, S)], ib[0], ib[2])
        pltpu.async_copy(dsti.at[pl.ds(base, S)], ib[1], ib[2])

    def _wait_idx(ib):
        pltpu.make_async_copy(srci.at[pl.ds(jnp.int32(0), S)], ib[0], ib[2]).wait()
        pltpu.make_async_copy(dsti.at[pl.ds(jnp.int32(0), S)], ib[1], ib[2]).wait()

    def _issue_rows(g, k, ib, rb):
        base = base0 + g * i32(C)
        kk = jnp.int32(k)
        pltpu.async_copy(dtab.at[ib[1].at[kk]], rb[0], rb[3])
        pltpu.async_copy(stab.at[ib[0].at[kk]], rb[1], rb[3])
        pltpu.async_copy(efx.at[pl.ds(base, C)], rb[2], rb[3])

    def _wait_rows(ib, rb):
        pltpu.make_async_copy(dtab.at[ib[1].at[jnp.int32(0)]], rb[0], rb[3]).wait()
        pltpu.make_async_copy(stab.at[ib[0].at[jnp.int32(0)]], rb[1], rb[3]).wait()
        pltpu.make_async_copy(efx.at[pl.ds(jnp.int32(0), C)], rb[2], rb[3]).wait()

    _issue_idx(i32(0), ibufs[0])
    _issue_idx(i32(1), ibufs[1])
    _wait_idx(ibufs[0])
    _issue_rows(i32(0), jnp.int32(0), ibufs[0], rbufs[0])

    lanes = lax.iota(jnp.int32, 16)
    pat01 = jnp.where(lanes < 8, jnp.int32(0), jnp.int32(1))
    pat23 = pat01 + jnp.int32(2)
    ILV = plsc.PackFormat.INTERLEAVED

    def _compute_chunk(rb, didx_row):
        drows, srows, efb = rb[0], rb[1], rb[2]

        def _edge(e, _):
            rowe = jnp.full((16,), e, jnp.int32)
            e0 = plsc.load_gather(efb, [rowe, pat01])
            e1 = plsc.load_gather(efb, [rowe, pat23])
            s = None
            for j in range(4):
                qa, qb = plsc.unpack(drows[e, pl.ds(32 * j, 32)], format=ILV)
                ka, kb = plsc.unpack(srows[e, pl.ds(32 * j, 32)], format=ILV)
                sj = qa * ka + qb * kb
                s = sj if s is None else s + sj
            qk = s + lax.rev(s, (0,))
            aa, ab = plsc.unpack(drows[e, pl.ds(128, 32)], format=ILV)
            t2 = aa * e0 + ab * e1
            et = t2 + lax.rev(t2, (0,))
            ex = jnp.exp((qk + et) * 0.25)
            for j in range(4):
                va, vb = plsc.unpack(srows[e, pl.ds(128 + 32 * j, 32)], format=ILV)
                msgm[e, pl.ds(32 * j, 16)] = va * ex
                msgm[e, pl.ds(32 * j + 16, 16)] = vb * ex
            msgs[e, pl.ds(0, 16)] = ex * e0
            msgs[e, pl.ds(16, 16)] = ex * e1
            msgs[e, pl.ds(32, 16)] = ex
            return _
        lax.fori_loop(jnp.int32(0), jnp.int32(C), _edge, 0)
        pltpu.sync_copy(msgm, accm.at[didx_row], add=True)
        pltpu.sync_copy(msgs, accs.at[didx_row], add=True)

    def _superpair(t, _):
        for u in range(2):
            m = t * i32(2) + i32(u)
            ib = ibufs[u]
            ibn = ibufs[1 - u]
            g0 = m * i32(S)
            for k in range(S):
                rb = rbufs[k & 1]
                rbn = rbufs[1 - (k & 1)]
                _wait_rows(ib, rb)
                if k < S - 1:
                    _issue_rows(g0 + i32(k + 1), jnp.int32(k + 1), ib, rbn)
                    _compute_chunk(rb, ib[1].at[jnp.int32(k)])
                else:
                    _wait_idx(ibn)
                    _issue_rows(g0 + i32(S), jnp.int32(0), ibn, rbn)
                    _compute_chunk(rb, ib[1].at[jnp.int32(k)])
                    _issue_idx(m + i32(2), ib)
        return _

    lax.fori_loop(jnp.int32(0), jnp.int32(NCH // (2 * S)), _superpair, 0)
    _wait_rows(ibufs[0], rbufs[0])
    _wait_idx(ibufs[1])
    plsc.subcore_barrier()

    for i in range(nfull):
        pltpu.sync_copy(accm.at[pl.ds(row0 + i * C, C)],
                        outm.at[cid, pl.ds(row0 + i * C, C)])
        pltpu.sync_copy(accs.at[pl.ds(row0 + i * C, C)],
                        outs.at[cid, pl.ds(row0 + i * C, C)])
    if rem:
        pltpu.sync_copy(accm.at[pl.ds(row0 + nfull * C, rem)],
                        outm.at[cid, pl.ds(row0 + nfull * C, rem)])
        pltpu.sync_copy(accs.at[pl.ds(row0 + nfull * C, rem)],
                        outs.at[cid, pl.ds(row0 + nfull * C, rem)])


_edge_pass = functools.partial(
    pl.kernel,
    out_type=(jax.ShapeDtypeStruct((2, NPAD, 128), jnp.float32),
              jax.ShapeDtypeStruct((2, NPAD, 48), jnp.float32)),
    mesh=plsc.VectorSubcoreMesh(core_axis_name="c", subcore_axis_name="s"),
    compiler_params=pltpu.CompilerParams(use_tc_tiling_on_sc=False,
                                        needs_layout_passes=False),
    scratch_types=[
        pltpu.VMEM_SHARED((NPAD, 128), jnp.float32),
        pltpu.VMEM_SHARED((NPAD, 48), jnp.float32),
        pltpu.VMEM((S, C), jnp.int32),
        pltpu.VMEM((S, C), jnp.int32),
        pltpu.VMEM((C, 160), jnp.bfloat16),
        pltpu.VMEM((C, 256), jnp.bfloat16),
        pltpu.VMEM((C, 4), jnp.float32),
        pltpu.VMEM((S, C), jnp.int32),
        pltpu.VMEM((S, C), jnp.int32),
        pltpu.VMEM((C, 160), jnp.bfloat16),
        pltpu.VMEM((C, 256), jnp.bfloat16),
        pltpu.VMEM((C, 4), jnp.float32),
        pltpu.VMEM((C, 128), jnp.float32),
        pltpu.VMEM((C, 48), jnp.float32),
        pltpu.SemaphoreType.DMA,
        pltpu.SemaphoreType.DMA,
        pltpu.SemaphoreType.DMA,
        pltpu.SemaphoreType.DMA,
    ],
)(_edge_kernel)


def _proj_body(accm_ref, accs_ref, wbig_ref, out_ref):
    accm = accm_ref[0] + accm_ref[1]
    accs = accs_ref[0] + accs_ref[1]
    den = accs[:, 32:48]
    rec = jnp.where(den > 0.0, 1.0 / den, 0.0)
    scaled = jnp.concatenate(
        [accm * jnp.concatenate([rec] * 8, axis=1),
         accs[:, :32] * jnp.concatenate([rec] * 2, axis=1)], axis=1)
    out_ref[...] = jnp.dot(scaled, wbig_ref[...],
                           preferred_element_type=jnp.float32,
                           precision=lax.Precision.HIGHEST)


def _mlp_body(out_ref, right_ref, g_ref, b_ref,
              w1a_ref, w1b_ref, b1_ref, w2t_ref, b2_ref, y_ref):
    out = out_ref[:N, :]
    mean = jnp.mean(out, axis=0, keepdims=True)
    dcen = out - mean
    var = jnp.mean(dcen * dcen, axis=0, keepdims=True)
    outb = dcen * lax.rsqrt(var + 1e-5) * g_ref[...] + b_ref[...]
    h1 = jnp.dot(outb, w1a_ref[...], preferred_element_type=jnp.float32,
                 precision=lax.Precision.HIGHEST)
    h1 = h1 + jnp.dot(right_ref[...], w1b_ref[...],
                      preferred_element_type=jnp.float32,
                      precision=lax.Precision.HIGHEST)
    h1 = jnp.maximum(h1 + b1_ref[...], 0.0)
    y_ref[...] = jnp.dot(h1, w2t_ref[...],
                         preferred_element_type=jnp.float32,
                         precision=lax.Precision.HIGHEST) + b2_ref[...]


def _epilogue(accm, accs, right, Wbig, gamma, beta, W1a, W1b, b1, W2t, b2):
    out = pl.pallas_call(
        _proj_body,
        grid=(NPAD // _RB,),
        in_specs=[pl.BlockSpec((2, _RB, 128), lambda i: (_Z, i, _Z)),
                  pl.BlockSpec((2, _RB, 48), lambda i: (_Z, i, _Z)),
                  pl.BlockSpec((160, D), lambda i: (_Z, _Z))],
        out_specs=pl.BlockSpec((_RB, D), lambda i: (i, _Z)),
        out_shape=jax.ShapeDtypeStruct((NPAD, D), jnp.float32),
    )(accm, accs, Wbig)
    return pl.pallas_call(
        _mlp_body,
        out_shape=jax.ShapeDtypeStruct((N, D), jnp.float32),
    )(out, right, gamma.reshape(1, D), beta.reshape(1, D),
      W1a, W1b, b1.reshape(1, D), W2t, b2.reshape(1, D))


def kernel(left_features, edge_indices, edge_features, right_features,
           Wq, Wk, Wv, Wke, Wve, Wout, bn_gamma, bn_beta, W1, b1, W2, b2):
    f32 = jnp.float32
    left = left_features.astype(f32)
    right = right_features.astype(f32)

    perm = _PERM
    Mq = jnp.einsum("hkd,hkc->dhc", Wq.reshape(H, DH, D), Wke.reshape(H, DH, ED))
    cols = []
    for p in range(2):
        for l in range(16):
            h = l if l < 8 else 15 - l
            c = 2 * p + (0 if l < 8 else 1)
            cols.append(Mq[:, h, c])
    M_lay = jnp.stack(cols, axis=1)
    Wd = jnp.concatenate([Wq.T[:, perm], M_lay], axis=1)[:, _ILV160]
    Ws = jnp.concatenate([Wk.T[:, perm], Wv.T[:, perm]], axis=1)[:, _ILV256]

    Gq = jnp.einsum("hkc,ohk->hco", Wve.reshape(H, DH, ED), Wout.reshape(D, H, DH))
    rows = []
    for p in range(2):
        for l in range(16):
            h = l if l < 8 else 15 - l
            c = 2 * p + (0 if l < 8 else 1)
            rows.append(Gq[h, c])
    G_lay = jnp.stack(rows, axis=0)
    Wbig = jnp.concatenate([Wout.T[perm], G_lay], axis=0)
    W1t = W1.T
    W1a, W1b = W1t[:D], W1t[D:]
    W2t = W2.T

    right_p = jnp.pad(right, ((0, NPAD - N), (0, 0)))
    left_p = jnp.pad(left, ((0, NPAD - N), (0, 0)))
    srci = jnp.full((EPAD2,), N, jnp.int32).at[:E].set(
        edge_indices[0].astype(jnp.int32)).reshape(-1, C)
    dsti = jnp.full((EPAD2,), N, jnp.int32).at[:E].set(
        edge_indices[1].astype(jnp.int32)).reshape(-1, C)
    ef = edge_features.astype(f32)
    efx = jnp.pad(ef, ((0, EPAD2 - E), (0, 0)))

    dtab, stab = _make_tables(right_p, left_p, Wd, Ws)
    accm, accs = _edge_pass(dtab, stab, efx, srci, dsti)
    y = _epilogue(accm, accs, right, Wbig,
                  bn_gamma.astype(f32), bn_beta.astype(f32),
                  W1a, W1b, b1.astype(f32), W2t, b2.astype(f32))
    return y.astype(jnp.float64)
